# baseline (speedup 1.0000x reference)
import functools

import jax
import jax.numpy as jnp
from jax import lax
from jax.experimental import pallas as pl
from jax.experimental.pallas import tpu as pltpu
from jax.experimental.pallas import tpu_sc as plsc

N = 10000
E = 320000
D = 128

NC = 2
NS = 16
NW = NC * NS
EPW = E // NW
CHUNK = 64
GK = 8
NG = 20
EPWP = NG * GK * CHUNK
NA = N + NS
RPS = 624
TAIL = N - NS * RPS

_sc_mesh = plsc.VectorSubcoreMesh(core_axis_name="c", subcore_axis_name="s")


@functools.partial(
    pl.kernel,
    out_type=jax.ShapeDtypeStruct((NC, N, D), jnp.float32),
    mesh=_sc_mesh,
    scratch_types=[
        pltpu.VMEM((2, GK, CHUNK), jnp.int32),
        pltpu.VMEM((2, GK, CHUNK), jnp.int32),
        pltpu.VMEM((CHUNK, D), jnp.float32),
        pltpu.VMEM((CHUNK, D), jnp.float32),
        pltpu.VMEM_SHARED((NA, D), jnp.float32),
        pltpu.SemaphoreType.DMA,
        pltpu.SemaphoreType.DMA,
        pltpu.SemaphoreType.DMA,
    ],
)
def _sc_aggregate(node_hbm, src_hbm, dst_hbm, out_hbm,
                  srcg, dstg, rows_a, rows_b, accum, sem_a, sem_b, isem):
    c = lax.axis_index("c")
    s = lax.axis_index("s")
    w = s * NC + c

    pltpu.sync_copy(node_hbm.at[pl.ds(s * RPS, RPS)],
                    accum.at[pl.ds(s * RPS, RPS)])

    @pl.when(s == 0)
    def _init_tail():
        pltpu.sync_copy(node_hbm.at[pl.ds(NS * RPS, TAIL)],
                        accum.at[pl.ds(NS * RPS, TAIL)])

    pltpu.async_copy(src_hbm.at[w, 0], srcg.at[0], isem)
    pltpu.async_copy(dst_hbm.at[w, 0], dstg.at[0], isem)
    plsc.subcore_barrier()

    def gather(idx_row, buf, sem):
        pltpu.async_copy(node_hbm.at[idx_row], buf, sem)

    def gather_wait(idx_row, buf, sem):
        pltpu.make_async_copy(node_hbm.at[idx_row], buf, sem).wait()

    def scatter(idx_row, buf):
        pltpu.sync_copy(buf, accum.at[idx_row], add=True)

    @pl.loop(0, NG)
    def _grp(g):
        q = lax.rem(g, 2)
        pltpu.make_async_copy(src_hbm.at[w, g], srcg.at[q], isem).wait()
        pltpu.make_async_copy(dst_hbm.at[w, g], dstg.at[q], isem).wait()

        @pl.when(g + 1 < NG)
        def _prefetch():
            pltpu.async_copy(src_hbm.at[w, g + 1], srcg.at[1 - q], isem)
            pltpu.async_copy(dst_hbm.at[w, g + 1], dstg.at[1 - q], isem)

        sv = srcg.at[q]
        dv = dstg.at[q]
        for j in range(GK):
            gather(sv.at[j], rows_a, sem_a)
            gather_wait(sv.at[j], rows_a, sem_a)
            scatter(dv.at[j], rows_a)

    plsc.subcore_barrier()
    pltpu.sync_copy(accum.at[pl.ds(s * RPS, RPS)],
                    out_hbm.at[c, pl.ds(s * RPS, RPS)])

    @pl.when(s == 0)
    def _out_tail():
        pltpu.sync_copy(accum.at[pl.ds(NS * RPS, TAIL)],
                        out_hbm.at[c, pl.ds(NS * RPS, TAIL)])


BLK = 1000


def _mlp_body(node_ref, p0_ref, p1_ref, eps_ref,
              w1_ref, b1_ref, g1_ref, be1_ref,
              w2_ref, b2_ref, g2_ref, be2_ref,
              w3_ref, b3_ref, gn_ref, bn_ref, o_ref):
    def ln(x, g, b):
        mu = jnp.mean(x, axis=-1, keepdims=True)
        var = jnp.mean((x - mu) ** 2, axis=-1, keepdims=True)
        return (x - mu) * lax.rsqrt(var + 1e-5) * g + b

    eps = eps_ref[0]
    h = p0_ref[0] + p1_ref[0] + (eps - 1.0) * node_ref[...]
    h = ln(jnp.dot(h, w1_ref[...], preferred_element_type=jnp.float32)
           + b1_ref[...], g1_ref[...], be1_ref[...])
    h = jnp.maximum(h, 0.0)
    h = ln(jnp.dot(h, w2_ref[...], preferred_element_type=jnp.float32)
           + b2_ref[...], g2_ref[...], be2_ref[...])
    h = jnp.maximum(h, 0.0)
    h = jnp.dot(h, w3_ref[...], preferred_element_type=jnp.float32) + b3_ref[...]
    o_ref[...] = jnp.maximum(ln(h, gn_ref[...], bn_ref[...]), 0.0)


_row_spec = pl.BlockSpec((BLK, D), lambda i: (i, 0))
_p_spec0 = pl.BlockSpec((1, BLK, D), lambda i: (0, i, 0))
_p_spec1 = pl.BlockSpec((1, BLK, D), lambda i: (1, i, 0))
_w_spec = pl.BlockSpec((D, D), lambda i: (0, 0))
_v_spec = pl.BlockSpec((1, D), lambda i: (0, 0))
_s_spec = pl.BlockSpec(memory_space=pltpu.SMEM)

_mlp_call = pl.pallas_call(
    _mlp_body,
    grid=(N // BLK,),
    in_specs=[_row_spec, _p_spec0, _p_spec1, _s_spec,
              _w_spec, _v_spec, _v_spec, _v_spec,
              _w_spec, _v_spec, _v_spec, _v_spec,
              _w_spec, _v_spec, _v_spec, _v_spec],
    out_specs=_row_spec,
    out_shape=jax.ShapeDtypeStruct((N, D), jnp.float32),
)


def kernel(node, edge_index, edge_attr, batch_ptr,
           W1, b1, g1, be1, W2, b2, g2, be2, W3, b3, eps, gN, bN):
    ei = edge_index.astype(jnp.int32)
    pad = EPWP - EPW
    src = jnp.pad(ei[0].reshape(NW, EPW), ((0, 0), (0, pad)),
                  constant_values=0).reshape(NW, NG, GK, CHUNK)
    junk = (N + jnp.arange(NW, dtype=jnp.int32) // NC)[:, None]
    dst = jnp.concatenate(
        [ei[1].reshape(NW, EPW),
         jnp.broadcast_to(junk, (NW, pad))], axis=1,
    ).reshape(NW, NG, GK, CHUNK)
    partials = _sc_aggregate(node, src, dst)
    eps1 = jnp.reshape(eps, (1,)).astype(jnp.float32)
    row = lambda v: jnp.reshape(v, (1, D))
    return _mlp_call(node, partials, partials, eps1,
                     W1, row(b1), row(g1), row(be1),
                     W2, row(b2), row(g2), row(be2),
                     W3, row(b3), row(gN), row(bN))


# speedup vs baseline: 1.0002x; 1.0002x over previous
import functools

import jax
import jax.numpy as jnp
from jax import lax
from jax.experimental import pallas as pl
from jax.experimental.pallas import tpu as pltpu
from jax.experimental.pallas import tpu_sc as plsc

N = 10000
E = 320000
D = 128

NC = 2
NS = 16
NW = NC * NS
EPW = E // NW
CHUNK = 64
GK = 8
NG = 20
EPWP = NG * GK * CHUNK
NA = N + NS
RPS = 624
TAIL = N - NS * RPS

_sc_mesh = plsc.VectorSubcoreMesh(core_axis_name="c", subcore_axis_name="s")


@functools.partial(
    pl.kernel,
    out_type=jax.ShapeDtypeStruct((NC, N, D), jnp.float32),
    mesh=_sc_mesh,
    scratch_types=[
        pltpu.VMEM((2 * GK, CHUNK), jnp.int32),
        pltpu.VMEM((2 * GK, CHUNK), jnp.int32),
        pltpu.VMEM((CHUNK, D), jnp.float32),
        pltpu.VMEM((CHUNK, D), jnp.float32),
        pltpu.VMEM_SHARED((NA, D), jnp.float32),
        pltpu.SemaphoreType.DMA,
        pltpu.SemaphoreType.DMA,
        pltpu.SemaphoreType.DMA,
    ],
)
def _sc_aggregate(node_hbm, src_hbm, dst_hbm, out_hbm,
                  srcg, dstg, rows_a, rows_b, accum, sem_a, sem_b, isem):
    c = lax.axis_index("c")
    s = lax.axis_index("s")
    w = s * NC + c

    pltpu.sync_copy(node_hbm.at[pl.ds(s * RPS, RPS)],
                    accum.at[pl.ds(s * RPS, RPS)])

    @pl.when(s == 0)
    def _init_tail():
        pltpu.sync_copy(node_hbm.at[pl.ds(NS * RPS, TAIL)],
                        accum.at[pl.ds(NS * RPS, TAIL)])

    pltpu.async_copy(src_hbm.at[w, 0], srcg.at[pl.ds(0, GK)], isem)
    pltpu.async_copy(dst_hbm.at[w, 0], dstg.at[pl.ds(0, GK)], isem)
    plsc.subcore_barrier()

    def gather(idx_row, buf, sem):
        pltpu.async_copy(node_hbm.at[idx_row], buf, sem)

    def gather_wait(idx_row, buf, sem):
        pltpu.make_async_copy(node_hbm.at[idx_row], buf, sem).wait()

    def scatter(idx_row, buf):
        pltpu.sync_copy(buf, accum.at[idx_row], add=True)

    @pl.loop(0, NG)
    def _grp(g):
        q = lax.rem(g, 2)
        pltpu.make_async_copy(src_hbm.at[w, g], srcg.at[pl.ds(q * GK, GK)], isem).wait()
        pltpu.make_async_copy(dst_hbm.at[w, g], dstg.at[pl.ds(q * GK, GK)], isem).wait()

        @pl.when(g + 1 < NG)
        def _prefetch():
            p = 1 - q
            pltpu.async_copy(src_hbm.at[w, g + 1], srcg.at[pl.ds(p * GK, GK)], isem)
            pltpu.async_copy(dst_hbm.at[w, g + 1], dstg.at[pl.ds(p * GK, GK)], isem)

        base = q * GK
        for j in range(GK):
            gather(srcg.at[base + j], rows_a, sem_a)
            gather_wait(srcg.at[base + j], rows_a, sem_a)
            scatter(dstg.at[base + j], rows_a)

    plsc.subcore_barrier()
    pltpu.sync_copy(accum.at[pl.ds(s * RPS, RPS)],
                    out_hbm.at[c, pl.ds(s * RPS, RPS)])

    @pl.when(s == 0)
    def _out_tail():
        pltpu.sync_copy(accum.at[pl.ds(NS * RPS, TAIL)],
                        out_hbm.at[c, pl.ds(NS * RPS, TAIL)])


BLK = 1000


def _mlp_body(node_ref, p0_ref, p1_ref, eps_ref,
              w1_ref, b1_ref, g1_ref, be1_ref,
              w2_ref, b2_ref, g2_ref, be2_ref,
              w3_ref, b3_ref, gn_ref, bn_ref, o_ref):
    def ln(x, g, b):
        mu = jnp.mean(x, axis=-1, keepdims=True)
        var = jnp.mean((x - mu) ** 2, axis=-1, keepdims=True)
        return (x - mu) * lax.rsqrt(var + 1e-5) * g + b

    eps = eps_ref[0]
    h = p0_ref[0] + p1_ref[0] + (eps - 1.0) * node_ref[...]
    h = ln(jnp.dot(h, w1_ref[...], preferred_element_type=jnp.float32)
           + b1_ref[...], g1_ref[...], be1_ref[...])
    h = jnp.maximum(h, 0.0)
    h = ln(jnp.dot(h, w2_ref[...], preferred_element_type=jnp.float32)
           + b2_ref[...], g2_ref[...], be2_ref[...])
    h = jnp.maximum(h, 0.0)
    h = jnp.dot(h, w3_ref[...], preferred_element_type=jnp.float32) + b3_ref[...]
    o_ref[...] = jnp.maximum(ln(h, gn_ref[...], bn_ref[...]), 0.0)


_row_spec = pl.BlockSpec((BLK, D), lambda i: (i, 0))
_p_spec0 = pl.BlockSpec((1, BLK, D), lambda i: (0, i, 0))
_p_spec1 = pl.BlockSpec((1, BLK, D), lambda i: (1, i, 0))
_w_spec = pl.BlockSpec((D, D), lambda i: (0, 0))
_v_spec = pl.BlockSpec((1, D), lambda i: (0, 0))
_s_spec = pl.BlockSpec(memory_space=pltpu.SMEM)

_mlp_call = pl.pallas_call(
    _mlp_body,
    grid=(N // BLK,),
    in_specs=[_row_spec, _p_spec0, _p_spec1, _s_spec,
              _w_spec, _v_spec, _v_spec, _v_spec,
              _w_spec, _v_spec, _v_spec, _v_spec,
              _w_spec, _v_spec, _v_spec, _v_spec],
    out_specs=_row_spec,
    out_shape=jax.ShapeDtypeStruct((N, D), jnp.float32),
)


def kernel(node, edge_index, edge_attr, batch_ptr,
           W1, b1, g1, be1, W2, b2, g2, be2, W3, b3, eps, gN, bN):
    ei = edge_index.astype(jnp.int32)
    pad = EPWP - EPW
    src = jnp.pad(ei[0].reshape(NW, EPW), ((0, 0), (0, pad)),
                  constant_values=0).reshape(NW, NG, GK, CHUNK)
    junk = (N + jnp.arange(NW, dtype=jnp.int32) // NC)[:, None]
    dst = jnp.concatenate(
        [ei[1].reshape(NW, EPW),
         jnp.broadcast_to(junk, (NW, pad))], axis=1,
    ).reshape(NW, NG, GK, CHUNK)
    partials = _sc_aggregate(node, src, dst)
    eps1 = jnp.reshape(eps, (1,)).astype(jnp.float32)
    row = lambda v: jnp.reshape(v, (1, D))
    return _mlp_call(node, partials, partials, eps1,
                     W1, row(b1), row(g1), row(be1),
                     W2, row(b2), row(g2), row(be2),
                     W3, row(b3), row(gN), row(bN))


# speedup vs baseline: 1.0011x; 1.0009x over previous
import functools

import jax
import jax.numpy as jnp
from jax import lax
from jax.experimental import pallas as pl
from jax.experimental.pallas import tpu as pltpu
from jax.experimental.pallas import tpu_sc as plsc

N = 10000
E = 320000
D = 128

NC = 2
NS = 16
NW = NC * NS
EPW = E // NW
CHUNK = 64
GK = 8
NG = 20
EPWP = NG * GK * CHUNK
NA = N + NS
RPS = 624
TAIL = N - NS * RPS

_sc_mesh = plsc.VectorSubcoreMesh(core_axis_name="c", subcore_axis_name="s")


@functools.partial(
    pl.kernel,
    out_type=jax.ShapeDtypeStruct((NC, N, D), jnp.float32),
    mesh=_sc_mesh,
    scratch_types=[
        pltpu.VMEM((2 * GK, CHUNK), jnp.int32),
        pltpu.VMEM((2 * GK, CHUNK), jnp.int32),
        pltpu.VMEM((CHUNK, D), jnp.float32),
        pltpu.VMEM((CHUNK, D), jnp.float32),
        pltpu.VMEM_SHARED((NA, D), jnp.float32),
        pltpu.SemaphoreType.DMA,
        pltpu.SemaphoreType.DMA,
        pltpu.SemaphoreType.DMA,
    ],
)
def _sc_aggregate(node_hbm, src_hbm, dst_hbm, out_hbm,
                  srcg, dstg, rows_a, rows_b, accum, sem_a, sem_b, isem):
    c = lax.axis_index("c")
    s = lax.axis_index("s")
    w = s * NC + c

    pltpu.sync_copy(node_hbm.at[pl.ds(s * RPS, RPS)],
                    accum.at[pl.ds(s * RPS, RPS)])

    @pl.when(s == 0)
    def _init_tail():
        pltpu.sync_copy(node_hbm.at[pl.ds(NS * RPS, TAIL)],
                        accum.at[pl.ds(NS * RPS, TAIL)])

    pltpu.async_copy(src_hbm.at[w, 0], srcg.at[pl.ds(0, GK)], isem)
    pltpu.async_copy(dst_hbm.at[w, 0], dstg.at[pl.ds(0, GK)], isem)
    plsc.subcore_barrier()

    def gather(idx_row, buf, sem):
        pltpu.async_copy(node_hbm.at[idx_row], buf, sem)

    def gather_wait(idx_row, buf, sem):
        pltpu.make_async_copy(node_hbm.at[idx_row], buf, sem).wait()

    def scatter(idx_row, buf):
        pltpu.sync_copy(buf, accum.at[idx_row], add=True)

    @pl.loop(0, NG)
    def _grp(g):
        q = lax.rem(g, 2)
        pltpu.make_async_copy(src_hbm.at[w, g], srcg.at[pl.ds(q * GK, GK)], isem).wait()
        pltpu.make_async_copy(dst_hbm.at[w, g], dstg.at[pl.ds(q * GK, GK)], isem).wait()

        @pl.when(g + 1 < NG)
        def _prefetch():
            p = 1 - q
            pltpu.async_copy(src_hbm.at[w, g + 1], srcg.at[pl.ds(p * GK, GK)], isem)
            pltpu.async_copy(dst_hbm.at[w, g + 1], dstg.at[pl.ds(p * GK, GK)], isem)

        base = q * GK
        for j in range(GK):
            cp = pltpu.async_copy(node_hbm.at[srcg.at[base + j]], rows_a, sem_a)
            cp.wait()
            scatter(dstg.at[base + j], rows_a)

    plsc.subcore_barrier()
    pltpu.sync_copy(accum.at[pl.ds(s * RPS, RPS)],
                    out_hbm.at[c, pl.ds(s * RPS, RPS)])

    @pl.when(s == 0)
    def _out_tail():
        pltpu.sync_copy(accum.at[pl.ds(NS * RPS, TAIL)],
                        out_hbm.at[c, pl.ds(NS * RPS, TAIL)])


BLK = 1000


def _mlp_body(node_ref, p0_ref, p1_ref, eps_ref,
              w1_ref, b1_ref, g1_ref, be1_ref,
              w2_ref, b2_ref, g2_ref, be2_ref,
              w3_ref, b3_ref, gn_ref, bn_ref, o_ref):
    def ln(x, g, b):
        mu = jnp.mean(x, axis=-1, keepdims=True)
        var = jnp.mean((x - mu) ** 2, axis=-1, keepdims=True)
        return (x - mu) * lax.rsqrt(var + 1e-5) * g + b

    eps = eps_ref[0]
    h = p0_ref[0] + p1_ref[0] + (eps - 1.0) * node_ref[...]
    h = ln(jnp.dot(h, w1_ref[...], preferred_element_type=jnp.float32)
           + b1_ref[...], g1_ref[...], be1_ref[...])
    h = jnp.maximum(h, 0.0)
    h = ln(jnp.dot(h, w2_ref[...], preferred_element_type=jnp.float32)
           + b2_ref[...], g2_ref[...], be2_ref[...])
    h = jnp.maximum(h, 0.0)
    h = jnp.dot(h, w3_ref[...], preferred_element_type=jnp.float32) + b3_ref[...]
    o_ref[...] = jnp.maximum(ln(h, gn_ref[...], bn_ref[...]), 0.0)


_row_spec = pl.BlockSpec((BLK, D), lambda i: (i, 0))
_p_spec0 = pl.BlockSpec((1, BLK, D), lambda i: (0, i, 0))
_p_spec1 = pl.BlockSpec((1, BLK, D), lambda i: (1, i, 0))
_w_spec = pl.BlockSpec((D, D), lambda i: (0, 0))
_v_spec = pl.BlockSpec((1, D), lambda i: (0, 0))
_s_spec = pl.BlockSpec(memory_space=pltpu.SMEM)

_mlp_call = pl.pallas_call(
    _mlp_body,
    grid=(N // BLK,),
    in_specs=[_row_spec, _p_spec0, _p_spec1, _s_spec,
              _w_spec, _v_spec, _v_spec, _v_spec,
              _w_spec, _v_spec, _v_spec, _v_spec,
              _w_spec, _v_spec, _v_spec, _v_spec],
    out_specs=_row_spec,
    out_shape=jax.ShapeDtypeStruct((N, D), jnp.float32),
)


def kernel(node, edge_index, edge_attr, batch_ptr,
           W1, b1, g1, be1, W2, b2, g2, be2, W3, b3, eps, gN, bN):
    ei = edge_index.astype(jnp.int32)
    pad = EPWP - EPW
    src = jnp.pad(ei[0].reshape(NW, EPW), ((0, 0), (0, pad)),
                  constant_values=0).reshape(NW, NG, GK, CHUNK)
    junk = (N + jnp.arange(NW, dtype=jnp.int32) // NC)[:, None]
    dst = jnp.concatenate(
        [ei[1].reshape(NW, EPW),
         jnp.broadcast_to(junk, (NW, pad))], axis=1,
    ).reshape(NW, NG, GK, CHUNK)
    partials = _sc_aggregate(node, src, dst)
    eps1 = jnp.reshape(eps, (1,)).astype(jnp.float32)
    row = lambda v: jnp.reshape(v, (1, D))
    return _mlp_call(node, partials, partials, eps1,
                     W1, row(b1), row(g1), row(be1),
                     W2, row(b2), row(g2), row(be2),
                     W3, row(b3), row(gN), row(bN))


# speedup vs baseline: 1.5862x; 1.5844x over previous
import functools

import jax
import jax.numpy as jnp
from jax import lax
from jax.experimental import pallas as pl
from jax.experimental.pallas import tpu as pltpu
from jax.experimental.pallas import tpu_sc as plsc

N = 10000
E = 320000
D = 128

NC = 2
NS = 16
NW = NC * NS
EPW = E // NW
CHUNK = 128
NCHUNK = 79
EPWP = NCHUNK * CHUNK
NA = N + NS
RPS = 624
TAIL = N - NS * RPS

_sc_mesh = plsc.VectorSubcoreMesh(core_axis_name="c", subcore_axis_name="s")


@functools.partial(
    pl.kernel,
    out_type=jax.ShapeDtypeStruct((NC, N, D), jnp.float32),
    mesh=_sc_mesh,
    scratch_types=[
        pltpu.VMEM((NCHUNK, CHUNK), jnp.int32),
        pltpu.VMEM((NCHUNK, CHUNK), jnp.int32),
        pltpu.VMEM((CHUNK, D), jnp.float32),
        pltpu.VMEM_SHARED((NA, D), jnp.float32),
        pltpu.SemaphoreType.DMA,
    ],
)
def _sc_aggregate(node_hbm, src_hbm, dst_hbm, out_hbm,
                  src_v, dst_v, rows_v, accum, sem_a):
    c = lax.axis_index("c")
    s = lax.axis_index("s")
    w = s * NC + c

    pltpu.sync_copy(node_hbm.at[pl.ds(s * RPS, RPS)],
                    accum.at[pl.ds(s * RPS, RPS)])

    @pl.when(s == 0)
    def _init_tail():
        pltpu.sync_copy(node_hbm.at[pl.ds(NS * RPS, TAIL)],
                        accum.at[pl.ds(NS * RPS, TAIL)])

    pltpu.sync_copy(src_hbm.at[w], src_v)
    pltpu.sync_copy(dst_hbm.at[w], dst_v)
    plsc.subcore_barrier()

    def body(i, carry):
        pltpu.async_copy(node_hbm.at[src_v.at[i]], rows_v, sem_a).wait()
        pltpu.sync_copy(rows_v, accum.at[dst_v.at[i]], add=True)
        return carry

    lax.fori_loop(0, NCHUNK, body, 0)

    plsc.subcore_barrier()
    pltpu.sync_copy(accum.at[pl.ds(s * RPS, RPS)],
                    out_hbm.at[c, pl.ds(s * RPS, RPS)])

    @pl.when(s == 0)
    def _out_tail():
        pltpu.sync_copy(accum.at[pl.ds(NS * RPS, TAIL)],
                        out_hbm.at[c, pl.ds(NS * RPS, TAIL)])


BLK = 1000


def _mlp_body(node_ref, p0_ref, p1_ref, eps_ref,
              w1_ref, b1_ref, g1_ref, be1_ref,
              w2_ref, b2_ref, g2_ref, be2_ref,
              w3_ref, b3_ref, gn_ref, bn_ref, o_ref):
    def ln(x, g, b):
        mu = jnp.mean(x, axis=-1, keepdims=True)
        var = jnp.mean((x - mu) ** 2, axis=-1, keepdims=True)
        return (x - mu) * lax.rsqrt(var + 1e-5) * g + b

    eps = eps_ref[0]
    h = p0_ref[0] + p1_ref[0] + (eps - 1.0) * node_ref[...]
    h = ln(jnp.dot(h, w1_ref[...], preferred_element_type=jnp.float32)
           + b1_ref[...], g1_ref[...], be1_ref[...])
    h = jnp.maximum(h, 0.0)
    h = ln(jnp.dot(h, w2_ref[...], preferred_element_type=jnp.float32)
           + b2_ref[...], g2_ref[...], be2_ref[...])
    h = jnp.maximum(h, 0.0)
    h = jnp.dot(h, w3_ref[...], preferred_element_type=jnp.float32) + b3_ref[...]
    o_ref[...] = jnp.maximum(ln(h, gn_ref[...], bn_ref[...]), 0.0)


_row_spec = pl.BlockSpec((BLK, D), lambda i: (i, 0))
_p_spec0 = pl.BlockSpec((1, BLK, D), lambda i: (0, i, 0))
_p_spec1 = pl.BlockSpec((1, BLK, D), lambda i: (1, i, 0))
_w_spec = pl.BlockSpec((D, D), lambda i: (0, 0))
_v_spec = pl.BlockSpec((1, D), lambda i: (0, 0))
_s_spec = pl.BlockSpec(memory_space=pltpu.SMEM)

_mlp_call = pl.pallas_call(
    _mlp_body,
    grid=(N // BLK,),
    in_specs=[_row_spec, _p_spec0, _p_spec1, _s_spec,
              _w_spec, _v_spec, _v_spec, _v_spec,
              _w_spec, _v_spec, _v_spec, _v_spec,
              _w_spec, _v_spec, _v_spec, _v_spec],
    out_specs=_row_spec,
    out_shape=jax.ShapeDtypeStruct((N, D), jnp.float32),
)


def kernel(node, edge_index, edge_attr, batch_ptr,
           W1, b1, g1, be1, W2, b2, g2, be2, W3, b3, eps, gN, bN):
    ei = edge_index.astype(jnp.int32)
    pad = EPWP - EPW
    src = jnp.pad(ei[0].reshape(NW, EPW), ((0, 0), (0, pad)),
                  constant_values=0).reshape(NW, NCHUNK, CHUNK)
    junk = (N + jnp.arange(NW, dtype=jnp.int32) // NC)[:, None]
    dst = jnp.concatenate(
        [ei[1].reshape(NW, EPW),
         jnp.broadcast_to(junk, (NW, pad))], axis=1,
    ).reshape(NW, NCHUNK, CHUNK)
    partials = _sc_aggregate(node, src, dst)
    eps1 = jnp.reshape(eps, (1,)).astype(jnp.float32)
    row = lambda v: jnp.reshape(v, (1, D))
    return _mlp_call(node, partials, partials, eps1,
                     W1, row(b1), row(g1), row(be1),
                     W2, row(b2), row(g2), row(be2),
                     W3, row(b3), row(gN), row(bN))


# speedup vs baseline: 3.3177x; 2.0917x over previous
import functools

import jax
import jax.numpy as jnp
from jax import lax
from jax.experimental import pallas as pl
from jax.experimental.pallas import tpu as pltpu
from jax.experimental.pallas import tpu_sc as plsc

N = 10000
E = 320000
D = 128

NC = 2
NS = 16
NW = NC * NS
EPW = E // NW
CHUNK = 80
NCHUNK = EPW // CHUNK
RPS = 624
TAIL = N - NS * RPS

_sc_mesh = plsc.VectorSubcoreMesh(core_axis_name="c", subcore_axis_name="s")


@functools.partial(
    pl.kernel,
    out_type=jax.ShapeDtypeStruct((NC, N, D), jnp.float32),
    mesh=_sc_mesh,
    scratch_types=[
        pltpu.VMEM((NCHUNK, CHUNK), jnp.int32),
        pltpu.VMEM((CHUNK,), jnp.int32),
        pltpu.VMEM((CHUNK,), jnp.int32),
        pltpu.VMEM((CHUNK,), jnp.int32),
        pltpu.VMEM((CHUNK,), jnp.int32),
        pltpu.VMEM((CHUNK, D), jnp.float32),
        pltpu.VMEM((CHUNK, D), jnp.float32),
        pltpu.VMEM_SHARED((N, D), jnp.float32),
        pltpu.SemaphoreType.DMA,
        pltpu.SemaphoreType.DMA,
    ],
)
def _sc_aggregate(node_hbm, comb_hbm, out_hbm,
                  comb_v, sa, da, sb, db, rows_a, rows_b,
                  accum, sem_a, sem_b):
    c = lax.axis_index("c")
    s = lax.axis_index("s")
    w = s * NC + c

    pltpu.sync_copy(node_hbm.at[pl.ds(s * RPS, RPS)],
                    accum.at[pl.ds(s * RPS, RPS)])

    @pl.when(s == 0)
    def _init_tail():
        pltpu.sync_copy(node_hbm.at[pl.ds(NS * RPS, TAIL)],
                        accum.at[pl.ds(NS * RPS, TAIL)])

    pltpu.sync_copy(comb_hbm.at[w], comb_v)
    plsc.subcore_barrier()

    def unpack(i, sbuf, dbuf):
        for k in range(CHUNK // 16):
            v = comb_v[i, pl.ds(k * 16, 16)]
            sbuf[pl.ds(k * 16, 16)] = jnp.minimum(v & 0x3FFF, N - 1)
            dbuf[pl.ds(k * 16, 16)] = jnp.minimum(
                lax.shift_right_logical(v, 14), N - 1)

    def gather(sbuf, buf, sem):
        pltpu.async_copy(node_hbm.at[sbuf], buf, sem)

    def gather_wait(sbuf, buf, sem):
        pltpu.make_async_copy(node_hbm.at[sbuf], buf, sem).wait()

    def scatter(dbuf, buf):
        pltpu.sync_copy(buf, accum.at[dbuf], add=True)

    unpack(0, sa, da)
    gather(sa, rows_a, sem_a)

    @pl.loop(0, NCHUNK, step=2)
    def _pair(g):
        @pl.when(g + 1 < NCHUNK)
        def _():
            unpack(g + 1, sb, db)
            gather(sb, rows_b, sem_b)

        gather_wait(sa, rows_a, sem_a)
        scatter(da, rows_a)

        @pl.when(g + 2 < NCHUNK)
        def _():
            unpack(g + 2, sa, da)
            gather(sa, rows_a, sem_a)

        @pl.when(g + 1 < NCHUNK)
        def _():
            gather_wait(sb, rows_b, sem_b)
            scatter(db, rows_b)

    plsc.subcore_barrier()
    pltpu.sync_copy(accum.at[pl.ds(s * RPS, RPS)],
                    out_hbm.at[c, pl.ds(s * RPS, RPS)])

    @pl.when(s == 0)
    def _out_tail():
        pltpu.sync_copy(accum.at[pl.ds(NS * RPS, TAIL)],
                        out_hbm.at[c, pl.ds(NS * RPS, TAIL)])


BLK = 1000


def _mlp_body(node_ref, p0_ref, p1_ref, eps_ref,
              w1_ref, b1_ref, g1_ref, be1_ref,
              w2_ref, b2_ref, g2_ref, be2_ref,
              w3_ref, b3_ref, gn_ref, bn_ref, o_ref):
    def ln(x, g, b):
        mu = jnp.mean(x, axis=-1, keepdims=True)
        var = jnp.mean((x - mu) ** 2, axis=-1, keepdims=True)
        return (x - mu) * lax.rsqrt(var + 1e-5) * g + b

    eps = eps_ref[0]
    h = p0_ref[0] + p1_ref[0] + (eps - 1.0) * node_ref[...]
    h = ln(jnp.dot(h, w1_ref[...], preferred_element_type=jnp.float32)
           + b1_ref[...], g1_ref[...], be1_ref[...])
    h = jnp.maximum(h, 0.0)
    h = ln(jnp.dot(h, w2_ref[...], preferred_element_type=jnp.float32)
           + b2_ref[...], g2_ref[...], be2_ref[...])
    h = jnp.maximum(h, 0.0)
    h = jnp.dot(h, w3_ref[...], preferred_element_type=jnp.float32) + b3_ref[...]
    o_ref[...] = jnp.maximum(ln(h, gn_ref[...], bn_ref[...]), 0.0)


_row_spec = pl.BlockSpec((BLK, D), lambda i: (i, 0))
_p_spec0 = pl.BlockSpec((1, BLK, D), lambda i: (0, i, 0))
_p_spec1 = pl.BlockSpec((1, BLK, D), lambda i: (1, i, 0))
_w_spec = pl.BlockSpec((D, D), lambda i: (0, 0))
_v_spec = pl.BlockSpec((1, D), lambda i: (0, 0))
_s_spec = pl.BlockSpec(memory_space=pltpu.SMEM)

_mlp_call = pl.pallas_call(
    _mlp_body,
    grid=(N // BLK,),
    in_specs=[_row_spec, _p_spec0, _p_spec1, _s_spec,
              _w_spec, _v_spec, _v_spec, _v_spec,
              _w_spec, _v_spec, _v_spec, _v_spec,
              _w_spec, _v_spec, _v_spec, _v_spec],
    out_specs=_row_spec,
    out_shape=jax.ShapeDtypeStruct((N, D), jnp.float32),
)


def kernel(node, edge_index, edge_attr, batch_ptr,
           W1, b1, g1, be1, W2, b2, g2, be2, W3, b3, eps, gN, bN):
    ei = edge_index.astype(jnp.int32)
    comb = (ei[0] + (ei[1] << 14)).reshape(NW, NCHUNK, CHUNK)
    partials = _sc_aggregate(node, comb)
    eps1 = jnp.reshape(eps, (1,)).astype(jnp.float32)
    row = lambda v: jnp.reshape(v, (1, D))
    return _mlp_call(node, partials, partials, eps1,
                     W1, row(b1), row(g1), row(be1),
                     W2, row(b2), row(g2), row(be2),
                     W3, row(b3), row(gN), row(bN))


# speedup vs baseline: 3.3830x; 1.0197x over previous
import functools

import jax
import jax.numpy as jnp
from jax import lax
from jax.experimental import pallas as pl
from jax.experimental.pallas import tpu as pltpu
from jax.experimental.pallas import tpu_sc as plsc

N = 10000
E = 320000
D = 128

NC = 2
NS = 16
NW = NC * NS
EPW = E // NW
CHUNK = 80
NCHUNK = EPW // CHUNK
RPS = 624
TAIL = N - NS * RPS

_sc_mesh = plsc.VectorSubcoreMesh(core_axis_name="c", subcore_axis_name="s")


@functools.partial(
    pl.kernel,
    out_type=jax.ShapeDtypeStruct((NC, N, D), jnp.float32),
    mesh=_sc_mesh,
    scratch_types=[
        pltpu.VMEM((NCHUNK, CHUNK), jnp.int32),
        pltpu.VMEM((CHUNK,), jnp.int32),
        pltpu.VMEM((CHUNK,), jnp.int32),
        pltpu.VMEM((CHUNK,), jnp.int32),
        pltpu.VMEM((CHUNK,), jnp.int32),
        pltpu.VMEM((CHUNK, D), jnp.float32),
        pltpu.VMEM((CHUNK, D), jnp.float32),
        pltpu.VMEM_SHARED((N, D), jnp.float32),
        pltpu.SemaphoreType.DMA,
        pltpu.SemaphoreType.DMA,
    ],
)
def _sc_aggregate(node_hbm, comb_hbm, out_hbm,
                  comb_v, sa, da, sb, db, rows_a, rows_b,
                  accum, sem_a, sem_b):
    c = lax.axis_index("c")
    s = lax.axis_index("s")
    w = s * NC + c

    pltpu.sync_copy(node_hbm.at[pl.ds(s * RPS, RPS)],
                    accum.at[pl.ds(s * RPS, RPS)])

    @pl.when(s == 0)
    def _init_tail():
        pltpu.sync_copy(node_hbm.at[pl.ds(NS * RPS, TAIL)],
                        accum.at[pl.ds(NS * RPS, TAIL)])

    pltpu.sync_copy(comb_hbm.at[w], comb_v)
    plsc.subcore_barrier()

    def unpack(i, sbuf, dbuf):
        for k in range(CHUNK // 16):
            v = comb_v[i, pl.ds(k * 16, 16)]
            sbuf[pl.ds(k * 16, 16)] = jnp.minimum(v & 0x3FFF, N - 1)
            dbuf[pl.ds(k * 16, 16)] = jnp.minimum(
                lax.shift_right_logical(v, 14), N - 1)

    def gather(sbuf, buf, sem):
        pltpu.async_copy(node_hbm.at[sbuf], buf, sem)

    def gather_wait(sbuf, buf, sem):
        pltpu.make_async_copy(node_hbm.at[sbuf], buf, sem).wait()

    def scatter(dbuf, buf):
        pltpu.sync_copy(buf, accum.at[dbuf], add=True)

    unpack(0, sa, da)
    gather(sa, rows_a, sem_a)

    @pl.loop(0, NCHUNK, step=2)
    def _pair(g):
        @pl.when(g + 1 < NCHUNK)
        def _():
            unpack(g + 1, sb, db)
            gather(sb, rows_b, sem_b)

        gather_wait(sa, rows_a, sem_a)
        scatter(da, rows_a)

        @pl.when(g + 2 < NCHUNK)
        def _():
            unpack(g + 2, sa, da)
            gather(sa, rows_a, sem_a)

        @pl.when(g + 1 < NCHUNK)
        def _():
            gather_wait(sb, rows_b, sem_b)
            scatter(db, rows_b)

    plsc.subcore_barrier()
    pltpu.sync_copy(accum.at[pl.ds(s * RPS, RPS)],
                    out_hbm.at[c, pl.ds(s * RPS, RPS)])

    @pl.when(s == 0)
    def _out_tail():
        pltpu.sync_copy(accum.at[pl.ds(NS * RPS, TAIL)],
                        out_hbm.at[c, pl.ds(NS * RPS, TAIL)])


BLK = 2000


def _mlp_body(node_ref, p0_ref, p1_ref, eps_ref,
              w1_ref, b1_ref, g1_ref, be1_ref,
              w2_ref, b2_ref, g2_ref, be2_ref,
              w3_ref, b3_ref, gn_ref, bn_ref, o_ref):
    def ln(x, g, b):
        mu = jnp.mean(x, axis=-1, keepdims=True)
        var = jnp.mean((x - mu) ** 2, axis=-1, keepdims=True)
        return (x - mu) * lax.rsqrt(var + 1e-5) * g + b

    eps = eps_ref[0]
    h = p0_ref[0] + p1_ref[0] + (eps - 1.0) * node_ref[...]
    h = ln(jnp.dot(h, w1_ref[...], preferred_element_type=jnp.float32)
           + b1_ref[...], g1_ref[...], be1_ref[...])
    h = jnp.maximum(h, 0.0)
    h = ln(jnp.dot(h, w2_ref[...], preferred_element_type=jnp.float32)
           + b2_ref[...], g2_ref[...], be2_ref[...])
    h = jnp.maximum(h, 0.0)
    h = jnp.dot(h, w3_ref[...], preferred_element_type=jnp.float32) + b3_ref[...]
    o_ref[...] = jnp.maximum(ln(h, gn_ref[...], bn_ref[...]), 0.0)


_row_spec = pl.BlockSpec((BLK, D), lambda i: (i, 0))
_p_spec0 = pl.BlockSpec((1, BLK, D), lambda i: (0, i, 0))
_p_spec1 = pl.BlockSpec((1, BLK, D), lambda i: (1, i, 0))
_w_spec = pl.BlockSpec((D, D), lambda i: (0, 0))
_v_spec = pl.BlockSpec((1, D), lambda i: (0, 0))
_s_spec = pl.BlockSpec(memory_space=pltpu.SMEM)

_mlp_call = pl.pallas_call(
    _mlp_body,
    grid=(N // BLK,),
    in_specs=[_row_spec, _p_spec0, _p_spec1, _s_spec,
              _w_spec, _v_spec, _v_spec, _v_spec,
              _w_spec, _v_spec, _v_spec, _v_spec,
              _w_spec, _v_spec, _v_spec, _v_spec],
    out_specs=_row_spec,
    out_shape=jax.ShapeDtypeStruct((N, D), jnp.float32),
)


def kernel(node, edge_index, edge_attr, batch_ptr,
           W1, b1, g1, be1, W2, b2, g2, be2, W3, b3, eps, gN, bN):
    ei = edge_index.astype(jnp.int32)
    comb = (ei[0] + (ei[1] << 14)).reshape(NW, NCHUNK, CHUNK)
    partials = _sc_aggregate(node, comb)
    eps1 = jnp.reshape(eps, (1,)).astype(jnp.float32)
    row = lambda v: jnp.reshape(v, (1, D))
    return _mlp_call(node, partials, partials, eps1,
                     W1, row(b1), row(g1), row(be1),
                     W2, row(b2), row(g2), row(be2),
                     W3, row(b3), row(gN), row(bN))
